# baseline (device time: 38084 ns/iter reference)
import jax
import jax.numpy as jnp
from jax import lax
from jax.experimental import pallas as pl
from jax.experimental.pallas import tpu as pltpu

H = 16
DH = 64
DR = 32
G = 4
HG = H // G
BF = jnp.bfloat16
F32 = jnp.float32
HC = HG * DH
RC = HG * DR


def _dot(a, b):
    return jnp.dot(a, b, preferred_element_type=F32)


def _dot_t(a, b):
    return lax.dot_general(a, b, (((1,), (1,)), ((), ())),
                           preferred_element_type=F32)


def kernel(x, Wdkv, Wuk, Wuv, Wq, Wqr, Wkr, Wo):
    B, S, D = x.shape
    dc = Wdkv.shape[1]
    scale = (DH + DR) ** -0.5

    def body(x_ref, wdkv_ref, wuk_ref, wuv_ref, wq_ref, wqr_ref, wkr_ref,
             wo_ref, out_ref,
             x16_ref, cs_ref, cr_ref,
             wuk16s_ref, wuv16s_ref, wukr_ref, wuvr_ref,
             wukm_ref, wuvm_ref, wukrm_ref, wuvrm_ref,
             q_ref, qr_ref, kr_ref, k16_ref, v16_ref,
             omy_ref, ofull_ref,
             wsend_sems, wrecv_sems, osend_sems, orecv_sems, copy_sem):
        my_x = lax.axis_index("x")
        my_y = lax.axis_index("y")
        my_z = lax.axis_index("z")
        partner = (1 - my_x, my_y, my_z)

        barrier = pltpu.get_barrier_semaphore()
        pl.semaphore_signal(barrier, inc=1, device_id=partner,
                            device_id_type=pl.DeviceIdType.MESH)
        for j in range(1, G):
            pl.semaphore_signal(barrier, inc=1,
                                device_id=(my_x, my_y, (my_z + j) % G),
                                device_id_type=pl.DeviceIdType.MESH)
        pl.semaphore_wait(barrier, G)

        wuk16s_ref[...] = wuk_ref[...].astype(BF)
        wuv16s_ref[...] = wuv_ref[...].astype(BF)
        wuk_rdma = pltpu.make_async_remote_copy(
            src_ref=wuk16s_ref, dst_ref=wukr_ref,
            send_sem=wsend_sems.at[0], recv_sem=wrecv_sems.at[0],
            device_id=partner, device_id_type=pl.DeviceIdType.MESH)
        wuk_rdma.start()
        wuv_rdma = pltpu.make_async_remote_copy(
            src_ref=wuv16s_ref, dst_ref=wuvr_ref,
            send_sem=wsend_sems.at[1], recv_sem=wrecv_sems.at[1],
            device_id=partner, device_id_type=pl.DeviceIdType.MESH)
        wuv_rdma.start()

        for b in range(B):
            x16_ref[b] = x_ref[b].astype(BF)
        for b in range(B):
            cs_ref[b] = _dot(x16_ref[b], wdkv_ref[...].astype(BF)).astype(BF)
        c_rdma = pltpu.make_async_remote_copy(
            src_ref=cs_ref, dst_ref=cr_ref,
            send_sem=wsend_sems.at[2], recv_sem=wrecv_sems.at[2],
            device_id=partner, device_id_type=pl.DeviceIdType.MESH)
        c_rdma.start()

        for g in range(G):
            @pl.when(my_z == g)
            def _(g=g):
                hc0 = g * HC
                rc0 = g * RC
                wq16 = wq_ref[:, hc0:hc0 + HC].astype(BF)
                wqr16 = wqr_ref[:, rc0:rc0 + RC].astype(BF)
                wukm_ref[...] = wuk16s_ref[:, hc0:hc0 + HC]
                wuvm_ref[...] = wuv16s_ref[:, hc0:hc0 + HC]
                for b in range(B):
                    q_ref[b] = (scale * _dot(x16_ref[b], wq16)).astype(BF)
                    qr_ref[b] = (scale * _dot(x16_ref[b], wqr16)).astype(BF)
        wkr16 = wkr_ref[...].astype(BF)
        for b in range(B):
            kr_ref[b] = _dot(x16_ref[b], wkr16).astype(BF)

        wuk_rdma.wait()
        wuv_rdma.wait()
        c_rdma.wait()

        for g in range(G):
            @pl.when(my_z == g)
            def _(g=g):
                hc0 = g * HC
                wukrm_ref[...] = wukr_ref[:, hc0:hc0 + HC]
                wuvrm_ref[...] = wuvr_ref[:, hc0:hc0 + HC]

        for b in range(B):
            k16_ref[b] = (_dot(cs_ref[b], wukm_ref[...])
                          + _dot(cr_ref[b], wukrm_ref[...])).astype(BF)
            v16_ref[b] = (_dot(cs_ref[b], wuvm_ref[...])
                          + _dot(cr_ref[b], wuvrm_ref[...])).astype(BF)

        for b in range(B):
            krb = kr_ref[b]
            for h in range(HG):
                qh = q_ref[b, :, h * DH:(h + 1) * DH]
                kh = k16_ref[b, :, h * DH:(h + 1) * DH]
                qrh = qr_ref[b, :, h * DR:(h + 1) * DR]
                s = _dot_t(qh, kh) + _dot_t(qrh, krb)
                p = jnp.exp(s)
                denom = jnp.sum(p, axis=-1, keepdims=True)
                oh = _dot(p.astype(BF), v16_ref[b, :, h * DH:(h + 1) * DH])
                omy_ref[b, :, h * DH:(h + 1) * DH] = (oh / denom).astype(BF)

        local_cp = pltpu.make_async_copy(
            omy_ref, ofull_ref.at[my_z], copy_sem)
        local_cp.start()
        o_rdmas = []
        for j in range(1, G):
            rdma = pltpu.make_async_remote_copy(
                src_ref=omy_ref, dst_ref=ofull_ref.at[my_z],
                send_sem=osend_sems.at[j - 1], recv_sem=orecv_sems.at[j - 1],
                device_id=(my_x, my_y, (my_z + j) % G),
                device_id_type=pl.DeviceIdType.MESH)
            rdma.start()
            o_rdmas.append(rdma)

        local_cp.wait()
        for m in range(G - 1):
            recv = pltpu.make_async_remote_copy(
                src_ref=omy_ref, dst_ref=ofull_ref.at[(my_z - m - 1) % G],
                send_sem=osend_sems.at[m], recv_sem=orecv_sems.at[m],
                device_id=partner, device_id_type=pl.DeviceIdType.MESH)
            recv.wait_recv()

        for b in range(B):
            acc = _dot(ofull_ref[0, b], wo_ref[0 * HC:1 * HC, :].astype(BF))
            for g in range(1, G):
                acc = acc + _dot(ofull_ref[g, b],
                                 wo_ref[g * HC:(g + 1) * HC, :].astype(BF))
            out_ref[b] = acc

        for rdma in o_rdmas:
            rdma.wait_send()

    return pl.pallas_call(
        body,
        out_shape=jax.ShapeDtypeStruct((B, S, D), F32),
        in_specs=[pl.BlockSpec(memory_space=pltpu.VMEM)] * 8,
        out_specs=pl.BlockSpec(memory_space=pltpu.VMEM),
        scratch_shapes=[
            pltpu.VMEM((B, S, D), BF),
            pltpu.VMEM((B, S, dc), BF),
            pltpu.VMEM((B, S, dc), BF),
            pltpu.VMEM(Wuk.shape, BF),
            pltpu.VMEM(Wuv.shape, BF),
            pltpu.VMEM(Wuk.shape, BF),
            pltpu.VMEM(Wuv.shape, BF),
            pltpu.VMEM((Wuk.shape[0], HC), BF),
            pltpu.VMEM((Wuv.shape[0], HC), BF),
            pltpu.VMEM((Wuk.shape[0], HC), BF),
            pltpu.VMEM((Wuv.shape[0], HC), BF),
            pltpu.VMEM((B, S, HC), BF),
            pltpu.VMEM((B, S, RC), BF),
            pltpu.VMEM((B, S, DR), BF),
            pltpu.VMEM((B, S, HC), BF),
            pltpu.VMEM((B, S, HC), BF),
            pltpu.VMEM((B, S, HC), BF),
            pltpu.VMEM((G, B, S, HC), BF),
            pltpu.SemaphoreType.DMA((3,)),
            pltpu.SemaphoreType.DMA((3,)),
            pltpu.SemaphoreType.DMA((G - 1,)),
            pltpu.SemaphoreType.DMA((G - 1,)),
            pltpu.SemaphoreType.DMA,
        ],
        compiler_params=pltpu.CompilerParams(collective_id=0),
    )(x, Wdkv, Wuk, Wuv, Wq, Wqr, Wkr, Wo)
